# baseline (speedup 1.0000x reference)
import functools

import jax
import jax.numpy as jnp
from jax.experimental import pallas as pl
from jax.experimental.pallas import tpu as pltpu

_E = 8
_TOPK = 2


def _moe_body(renorm_ref, x_ref, lg_ref, w1_ref, w3_ref, w2_ref, out_ref,
              acc_ref, *, n_e, n_i):
    e = pl.program_id(1)
    i = pl.program_id(2)

    @pl.when((e == 0) & (i == 0))
    def _():
        acc_ref[...] = jnp.zeros_like(acc_ref)

    x = x_ref[...].astype(jnp.bfloat16)
    w1 = w1_ref[0].astype(jnp.bfloat16)
    w3 = w3_ref[0].astype(jnp.bfloat16)
    w2 = w2_ref[0].astype(jnp.bfloat16)

    dn = (((1,), (1,)), ((), ()))
    gate = jax.lax.dot_general(x, w1, dn, preferred_element_type=jnp.float32)
    up = jax.lax.dot_general(x, w3, dn, preferred_element_type=jnp.float32)
    act = (gate * jax.lax.logistic(gate) * up).astype(jnp.bfloat16)
    dn2 = (((1,), (1,)), ((), ()))
    contrib = jax.lax.dot_general(act, w2, dn2,
                                  preferred_element_type=jnp.float32)

    lg = lg_ref[...].astype(jnp.float32)
    mx = jnp.max(lg, axis=-1, keepdims=True)
    ex = jnp.exp(lg - mx)
    sm = ex / jnp.sum(ex, axis=-1, keepdims=True)
    iota = jax.lax.broadcasted_iota(jnp.int32, sm.shape, 1)
    p1 = jnp.max(sm, axis=-1, keepdims=True)
    i1 = jnp.min(jnp.where(sm == p1, iota, n_e), axis=-1, keepdims=True)
    sm2 = jnp.where(iota == i1, -jnp.inf, sm)
    p2 = jnp.max(sm2, axis=-1, keepdims=True)
    i2 = jnp.min(jnp.where(sm2 == p2, iota, n_e), axis=-1, keepdims=True)
    denom = jnp.where(renorm_ref[0, 0] > 0.0, p1 + p2, 1.0)
    w_e = jnp.where(i1 == e, p1, jnp.where(i2 == e, p2, 0.0)) / denom

    acc_ref[...] += w_e * contrib

    @pl.when((e == n_e - 1) & (i == n_i - 1))
    def _():
        out_ref[...] = acc_ref[...]


def kernel(hidden_states, router_logits, W13, W2, use_grouped_topk, top_k,
           renormalize):
    T, H = hidden_states.shape
    E = W13.shape[0]
    I = W2.shape[2]
    BT = 256
    CI = 512
    n_t, n_i = T // BT, I // CI

    renorm = jnp.where(renormalize, 1.0, 0.0).astype(jnp.float32).reshape(1, 1)

    grid = (n_t, E, n_i)
    out = pl.pallas_call(
        functools.partial(_moe_body, n_e=E, n_i=n_i),
        grid=grid,
        in_specs=[
            pl.BlockSpec(memory_space=pltpu.SMEM),
            pl.BlockSpec((BT, H), lambda t, e, i: (t, 0)),
            pl.BlockSpec((BT, E), lambda t, e, i: (t, 0)),
            pl.BlockSpec((1, CI, H), lambda t, e, i: (e, i, 0)),
            pl.BlockSpec((1, CI, H), lambda t, e, i, _ni=I // CI: (e, _ni + i, 0)),
            pl.BlockSpec((1, H, CI), lambda t, e, i: (e, 0, i)),
        ],
        out_specs=pl.BlockSpec((BT, H), lambda t, e, i: (t, 0)),
        out_shape=jax.ShapeDtypeStruct((T, H), jnp.float32),
        scratch_shapes=[pltpu.VMEM((BT, H), jnp.float32)],
    )(renorm, hidden_states, router_logits, W13, W13, W2)

    out = out + jnp.where(use_grouped_topk, jnp.nan, 0.0)
    _ = top_k
    return out.reshape(-1, H)


# speedup vs baseline: 1.0043x; 1.0043x over previous
import functools

import jax
import jax.numpy as jnp
from jax.experimental import pallas as pl
from jax.experimental.pallas import tpu as pltpu

_E = 8
_TOPK = 2


def _moe_body(renorm_ref, x_ref, lg_ref, w1_ref, w3_ref, w2_ref, out_ref,
              acc_ref, *, n_e, n_i):
    e = pl.program_id(1)
    i = pl.program_id(2)

    @pl.when((e == 0) & (i == 0))
    def _():
        acc_ref[...] = jnp.zeros_like(acc_ref)

    x = x_ref[...]
    w1 = w1_ref[0]
    w3 = w3_ref[0]
    w2 = w2_ref[0]

    dn = (((1,), (1,)), ((), ()))
    gate = jax.lax.dot_general(x, w1, dn, preferred_element_type=jnp.float32)
    up = jax.lax.dot_general(x, w3, dn, preferred_element_type=jnp.float32)
    act = gate * jax.lax.logistic(gate) * up
    dn2 = (((1,), (1,)), ((), ()))
    contrib = jax.lax.dot_general(act, w2, dn2,
                                  preferred_element_type=jnp.float32)

    lg = lg_ref[...].astype(jnp.float32)
    mx = jnp.max(lg, axis=-1, keepdims=True)
    ex = jnp.exp(lg - mx)
    sm = ex / jnp.sum(ex, axis=-1, keepdims=True)
    iota = jax.lax.broadcasted_iota(jnp.int32, sm.shape, 1)
    p1 = jnp.max(sm, axis=-1, keepdims=True)
    i1 = jnp.min(jnp.where(sm == p1, iota, n_e), axis=-1, keepdims=True)
    sm2 = jnp.where(iota == i1, -jnp.inf, sm)
    p2 = jnp.max(sm2, axis=-1, keepdims=True)
    i2 = jnp.min(jnp.where(sm2 == p2, iota, n_e), axis=-1, keepdims=True)
    denom = jnp.where(renorm_ref[0, 0] > 0.0, p1 + p2, 1.0)
    w_e = jnp.where(i1 == e, p1, jnp.where(i2 == e, p2, 0.0)) / denom

    acc_ref[...] += w_e * contrib

    @pl.when((e == n_e - 1) & (i == n_i - 1))
    def _():
        out_ref[...] = acc_ref[...]


def kernel(hidden_states, router_logits, W13, W2, use_grouped_topk, top_k,
           renormalize):
    T, H = hidden_states.shape
    E = W13.shape[0]
    I = W2.shape[2]
    BT = 256
    CI = 512
    n_t, n_i = T // BT, I // CI

    renorm = jnp.where(renormalize, 1.0, 0.0).astype(jnp.float32).reshape(1, 1)

    grid = (n_t, E, n_i)
    out = pl.pallas_call(
        functools.partial(_moe_body, n_e=E, n_i=n_i),
        grid=grid,
        in_specs=[
            pl.BlockSpec(memory_space=pltpu.SMEM),
            pl.BlockSpec((BT, H), lambda t, e, i: (t, 0)),
            pl.BlockSpec((BT, E), lambda t, e, i: (t, 0)),
            pl.BlockSpec((1, CI, H), lambda t, e, i: (e, i, 0)),
            pl.BlockSpec((1, CI, H), lambda t, e, i, _ni=I // CI: (e, _ni + i, 0)),
            pl.BlockSpec((1, H, CI), lambda t, e, i: (e, 0, i)),
        ],
        out_specs=pl.BlockSpec((BT, H), lambda t, e, i: (t, 0)),
        out_shape=jax.ShapeDtypeStruct((T, H), jnp.float32),
        scratch_shapes=[pltpu.VMEM((BT, H), jnp.float32)],
    )(renorm, hidden_states, router_logits, W13, W13, W2)

    out = out + jnp.where(use_grouped_topk, jnp.nan, 0.0)
    _ = top_k
    return out.reshape(-1, H)


# speedup vs baseline: 1.0069x; 1.0026x over previous
import functools

import jax
import jax.numpy as jnp
from jax.experimental import pallas as pl
from jax.experimental.pallas import tpu as pltpu

_E = 8
_TOPK = 2


def _moe_body(renorm_ref, x_ref, lg_ref, w1_ref, w3_ref, w2_ref, out_ref,
              acc_ref, tw_ref, *, n_e, n_i):
    e = pl.program_id(1)
    i = pl.program_id(2)

    @pl.when((e == 0) & (i == 0))
    def _():
        acc_ref[...] = jnp.zeros_like(acc_ref)
        lg = lg_ref[...].astype(jnp.float32)
        mx = jnp.max(lg, axis=-1, keepdims=True)
        ex = jnp.exp(lg - mx)
        sm = ex / jnp.sum(ex, axis=-1, keepdims=True)
        iota = jax.lax.broadcasted_iota(jnp.int32, sm.shape, 1)
        p1 = jnp.max(sm, axis=-1, keepdims=True)
        i1 = jnp.min(jnp.where(sm == p1, iota, n_e), axis=-1, keepdims=True)
        sm2 = jnp.where(iota == i1, -jnp.inf, sm)
        p2 = jnp.max(sm2, axis=-1, keepdims=True)
        i2 = jnp.min(jnp.where(sm2 == p2, iota, n_e), axis=-1, keepdims=True)
        denom = jnp.where(renorm_ref[0, 0] > 0.0, p1 + p2, 1.0)
        tw_ref[...] = jnp.where(
            i1 == iota, p1, jnp.where(i2 == iota, p2, 0.0)) / denom

    x = x_ref[...]
    w1 = w1_ref[0]
    w3 = w3_ref[0]
    w2 = w2_ref[0]

    dn = (((1,), (1,)), ((), ()))
    gate = jax.lax.dot_general(x, w1, dn, preferred_element_type=jnp.float32)
    up = jax.lax.dot_general(x, w3, dn, preferred_element_type=jnp.float32)
    act = gate * jax.lax.logistic(gate) * up
    dn2 = (((1,), (1,)), ((), ()))
    contrib = jax.lax.dot_general(act, w2, dn2,
                                  preferred_element_type=jnp.float32)

    tw = tw_ref[...]
    iota = jax.lax.broadcasted_iota(jnp.int32, tw.shape, 1)
    w_e = jnp.sum(jnp.where(iota == e, tw, 0.0), axis=-1, keepdims=True)

    acc_ref[...] += w_e * contrib

    @pl.when((e == n_e - 1) & (i == n_i - 1))
    def _():
        out_ref[...] = acc_ref[...]


def kernel(hidden_states, router_logits, W13, W2, use_grouped_topk, top_k,
           renormalize):
    T, H = hidden_states.shape
    E = W13.shape[0]
    I = W2.shape[2]
    BT = 256
    CI = 512
    n_t, n_i = T // BT, I // CI

    renorm = jnp.where(renormalize, 1.0, 0.0).astype(jnp.float32).reshape(1, 1)

    grid = (n_t, E, n_i)
    out = pl.pallas_call(
        functools.partial(_moe_body, n_e=E, n_i=n_i),
        grid=grid,
        in_specs=[
            pl.BlockSpec(memory_space=pltpu.SMEM),
            pl.BlockSpec((BT, H), lambda t, e, i: (t, 0)),
            pl.BlockSpec((BT, E), lambda t, e, i: (t, 0)),
            pl.BlockSpec((1, CI, H), lambda t, e, i: (e, i, 0)),
            pl.BlockSpec((1, CI, H), lambda t, e, i, _ni=I // CI: (e, _ni + i, 0)),
            pl.BlockSpec((1, H, CI), lambda t, e, i: (e, 0, i)),
        ],
        out_specs=pl.BlockSpec((BT, H), lambda t, e, i: (t, 0)),
        out_shape=jax.ShapeDtypeStruct((T, H), jnp.float32),
        scratch_shapes=[pltpu.VMEM((BT, H), jnp.float32),
                        pltpu.VMEM((BT, _E), jnp.float32)],
    )(renorm, hidden_states, router_logits, W13, W13, W2)

    out = out + jnp.where(use_grouped_topk, jnp.nan, 0.0)
    _ = top_k
    return out.reshape(-1, H)


# speedup vs baseline: 1.7058x; 1.6942x over previous
import functools

import jax
import jax.numpy as jnp
from jax.experimental import pallas as pl
from jax.experimental.pallas import tpu as pltpu

_E = 8


def _routing_weights(lg, renorm_flag, n_e):
    lg = lg.astype(jnp.float32)
    mx = jnp.max(lg, axis=-1, keepdims=True)
    ex = jnp.exp(lg - mx)
    sm = ex / jnp.sum(ex, axis=-1, keepdims=True)
    iota = jax.lax.broadcasted_iota(jnp.int32, sm.shape, 1)
    p1 = jnp.max(sm, axis=-1, keepdims=True)
    i1 = jnp.min(jnp.where(sm == p1, iota, n_e), axis=-1, keepdims=True)
    sm2 = jnp.where(iota == i1, -jnp.inf, sm)
    p2 = jnp.max(sm2, axis=-1, keepdims=True)
    i2 = jnp.min(jnp.where(sm2 == p2, iota, n_e), axis=-1, keepdims=True)
    denom = jnp.where(renorm_flag > 0.0, p1 + p2, 1.0)
    return jnp.where(i1 == iota, p1, jnp.where(i2 == iota, p2, 0.0)) / denom


def _moe_body(renorm_ref, lg_ref, x_ref, w1_ref, w3_ref, w2_ref, out_ref,
              tw_ref, *, n_e, n_i, bt):
    e = pl.program_id(0)
    i = pl.program_id(1)
    t = pl.program_id(2)

    @pl.when((e == 0) & (i == 0) & (t == 0))
    def _():
        out_ref[...] = jnp.zeros_like(out_ref)
        tw_ref[...] = _routing_weights(lg_ref[...], renorm_ref[0, 0], n_e)

    x = x_ref[...]
    w1 = w1_ref[0]
    w3 = w3_ref[0]
    w2 = w2_ref[0]

    dn = (((1,), (1,)), ((), ()))
    gate = jax.lax.dot_general(x, w1, dn, preferred_element_type=jnp.float32)
    up = jax.lax.dot_general(x, w3, dn, preferred_element_type=jnp.float32)
    act = gate * jax.lax.logistic(gate) * up
    contrib = jax.lax.dot_general(act, w2, dn,
                                  preferred_element_type=jnp.float32)

    tw = tw_ref[pl.ds(t * bt, bt), :]
    iota = jax.lax.broadcasted_iota(jnp.int32, tw.shape, 1)
    w_e = jnp.sum(jnp.where(iota == e, tw, 0.0), axis=-1, keepdims=True)

    out_ref[pl.ds(t * bt, bt), :] += w_e * contrib


def kernel(hidden_states, router_logits, W13, W2, use_grouped_topk, top_k,
           renormalize):
    T, H = hidden_states.shape
    E = W13.shape[0]
    I = W2.shape[2]
    BT = 256
    CI = 1024
    n_t, n_i = T // BT, I // CI

    renorm = jnp.where(renormalize, 1.0, 0.0).astype(jnp.float32).reshape(1, 1)

    grid = (E, n_i, n_t)
    out = pl.pallas_call(
        functools.partial(_moe_body, n_e=E, n_i=n_i, bt=BT),
        grid=grid,
        in_specs=[
            pl.BlockSpec(memory_space=pltpu.SMEM),
            pl.BlockSpec((T, E), lambda e, i, t: (0, 0)),
            pl.BlockSpec((BT, H), lambda e, i, t: (t, 0)),
            pl.BlockSpec((1, CI, H), lambda e, i, t: (e, i, 0)),
            pl.BlockSpec((1, CI, H), lambda e, i, t, _ni=I // CI: (e, _ni + i, 0)),
            pl.BlockSpec((1, H, CI), lambda e, i, t: (e, 0, i)),
        ],
        out_specs=pl.BlockSpec((T, H), lambda e, i, t: (0, 0)),
        out_shape=jax.ShapeDtypeStruct((T, H), jnp.float32),
        scratch_shapes=[pltpu.VMEM((T, _E), jnp.float32)],
    )(renorm, router_logits, hidden_states, W13, W13, W2)

    out = out + jnp.where(use_grouped_topk, jnp.nan, 0.0)
    _ = top_k
    return out.reshape(-1, H)
